# Initial kernel scaffold; baseline (speedup 1.0000x reference)
import jax
import jax.numpy as jnp
from jax.experimental import pallas as pl


def kernel(x, edge_index, W1, ln1_w, ln1_b, W2, ln2_w, ln2_b, gn_weight, gn_bias, gn_mean_scale):
    raise NotImplementedError("write your pallas kernel here")


# speedup vs baseline: 3.4405x; 3.4405x over previous
import functools

import jax
import jax.numpy as jnp
from jax import lax
from jax.experimental import pallas as pl
from jax.experimental.pallas import tpu as pltpu
from jax.experimental.pallas import tpu_sc as plsc

N_NODES = 10000
D = 128
N_EDGES = 320000
LN_EPS = 1e-5
GN_EPS = 1e-5

NC = 2
NS = 16
NW = NC * NS
CHUNK = 128
K = 80
E_PAD = NW * K * CHUNK
N_PAD = 10112
ROWS_PER_TILE = N_PAD // NS

_mesh = plsc.VectorSubcoreMesh(core_axis_name="c", subcore_axis_name="s")


@functools.partial(
    pl.kernel,
    out_type=jax.ShapeDtypeStruct((NC, N_PAD, D), jnp.float32),
    mesh=_mesh,
    scratch_types=[
        pltpu.VMEM((K, CHUNK), jnp.int32),
        pltpu.VMEM((K, CHUNK), jnp.int32),
        pltpu.VMEM((CHUNK, D), jnp.float32),
        pltpu.VMEM_SHARED((N_PAD, D), jnp.float32),
        pltpu.SemaphoreType.DMA,
    ],
)
def _segment_sum_sc(x_hbm, src_hbm, dst_hbm, zeros_hbm, out_hbm,
                    src_v, dst_v, rows_v, agg_sh, sem):
    c = lax.axis_index("c")
    s = lax.axis_index("s")
    w = c * NS + s
    pltpu.sync_copy(zeros_hbm, agg_sh.at[pl.ds(s * ROWS_PER_TILE, ROWS_PER_TILE)])
    pltpu.sync_copy(src_hbm.at[pl.ds(w * K, K)], src_v)
    pltpu.sync_copy(dst_hbm.at[pl.ds(w * K, K)], dst_v)
    plsc.subcore_barrier()

    def body(j, carry):
        pltpu.async_copy(x_hbm.at[src_v.at[j]], rows_v, sem).wait()
        pltpu.sync_copy(rows_v, agg_sh.at[dst_v.at[j]], add=True)
        return carry

    lax.fori_loop(0, K, body, 0)
    plsc.subcore_barrier()
    pltpu.sync_copy(agg_sh.at[pl.ds(s * ROWS_PER_TILE, ROWS_PER_TILE)],
                    out_hbm.at[c, pl.ds(s * ROWS_PER_TILE, ROWS_PER_TILE)])


def _ln(h, w, b):
    mu = jnp.mean(h, axis=1, keepdims=True)
    d_ = h - mu
    var = jnp.mean(d_ * d_, axis=1, keepdims=True)
    return d_ * lax.rsqrt(var + LN_EPS) * w + b


def _dense_body(x_ref, agg_ref, w1t_ref, ln1w_ref, ln1b_ref, w2t_ref,
                ln2w_ref, ln2b_ref, gnw_ref, gnb_ref, gnms_ref, out_ref):
    h = x_ref[...] + agg_ref[0, :N_NODES, :] + agg_ref[1, :N_NODES, :]
    h = jnp.dot(h, w1t_ref[...], preferred_element_type=jnp.float32)
    h = jnp.maximum(_ln(h, ln1w_ref[...], ln1b_ref[...]), 0.0)
    h = jnp.dot(h, w2t_ref[...], preferred_element_type=jnp.float32)
    h = jnp.maximum(_ln(h, ln2w_ref[...], ln2b_ref[...]), 0.0)
    mean = jnp.mean(h, axis=0, keepdims=True)
    o = h - gnms_ref[...] * mean
    var = jnp.mean(o * o, axis=0, keepdims=True)
    out_ref[...] = o * lax.rsqrt(var + GN_EPS) * gnw_ref[...] + gnb_ref[...]


_dense = pl.pallas_call(
    _dense_body,
    out_shape=jax.ShapeDtypeStruct((N_NODES, D), jnp.float32),
)


def kernel(x, edge_index, W1, ln1_w, ln1_b, W2, ln2_w, ln2_b,
           gn_weight, gn_bias, gn_mean_scale):
    xp = jnp.zeros((N_PAD, D), jnp.float32).at[:N_NODES].set(x)
    pad = jnp.full((E_PAD - N_EDGES,), N_NODES, jnp.int32)
    srcp = jnp.concatenate([edge_index[0], pad]).reshape(NW * K, CHUNK)
    dstp = jnp.concatenate([edge_index[1], pad]).reshape(NW * K, CHUNK)
    zeros = jnp.zeros((ROWS_PER_TILE, D), jnp.float32)
    agg2 = _segment_sum_sc(xp, srcp, dstp, zeros)
    r = lambda v: v.reshape(1, D)
    return _dense(x, agg2, W1.T, r(ln1_w), r(ln1_b), W2.T, r(ln2_w),
                  r(ln2_b), r(gn_weight), r(gn_bias), r(gn_mean_scale))


# speedup vs baseline: 5.5143x; 1.6028x over previous
import functools

import jax
import jax.numpy as jnp
from jax import lax
from jax.experimental import pallas as pl
from jax.experimental.pallas import tpu as pltpu
from jax.experimental.pallas import tpu_sc as plsc

N_NODES = 10000
D = 128
DH = D // 2
N_EDGES = 320000
LN_EPS = 1e-5
GN_EPS = 1e-5

NC = 2
NS = 16
CHUNK = 128
K = 160
E_PAD = NS * K * CHUNK
N_PAD = 10112
ROWS_PER_TILE = N_PAD // NS
NBUF = 4

_mesh = plsc.VectorSubcoreMesh(core_axis_name="c", subcore_axis_name="s")


@functools.partial(
    pl.kernel,
    out_type=jax.ShapeDtypeStruct((NC, N_PAD, DH), jnp.float32),
    mesh=_mesh,
    compiler_params=pltpu.CompilerParams(use_tc_tiling_on_sc=False),
    scratch_types=[
        pltpu.VMEM((K, CHUNK), jnp.int32),
        pltpu.VMEM((K, CHUNK), jnp.int32),
        [pltpu.VMEM((CHUNK, DH), jnp.float32)] * NBUF,
        pltpu.VMEM_SHARED((N_PAD, DH), jnp.float32),
        pltpu.SemaphoreType.DMA,
    ],
)
def _segment_sum_sc(x2_hbm, src_hbm, dst_hbm, zeros_hbm, out_hbm,
                    src_v, dst_v, rows, agg_sh, sem):
    c = lax.axis_index("c")
    s = lax.axis_index("s")
    pltpu.sync_copy(zeros_hbm, agg_sh.at[pl.ds(s * ROWS_PER_TILE, ROWS_PER_TILE)])
    pltpu.sync_copy(src_hbm.at[c, pl.ds(s * K, K)], src_v)
    pltpu.sync_copy(dst_hbm.at[pl.ds(s * K, K)], dst_v)
    plsc.subcore_barrier()

    for b in range(NBUF):
        pltpu.async_copy(x2_hbm.at[src_v.at[b]], rows[b], sem)

    def body(g, carry):
        j0 = g * NBUF
        for b in range(NBUF):
            j = j0 + b
            pltpu.make_async_copy(x2_hbm.at[src_v.at[j]], rows[b], sem).wait()
            pltpu.sync_copy(rows[b], agg_sh.at[dst_v.at[j]], add=True)

            @pl.when(j + NBUF < K)
            def _():
                pltpu.async_copy(x2_hbm.at[src_v.at[j + NBUF]], rows[b], sem)
        return carry

    lax.fori_loop(0, K // NBUF, body, 0)
    plsc.subcore_barrier()
    pltpu.sync_copy(agg_sh.at[pl.ds(s * ROWS_PER_TILE, ROWS_PER_TILE)],
                    out_hbm.at[c, pl.ds(s * ROWS_PER_TILE, ROWS_PER_TILE)])


def _ln(h, w, b):
    mu = jnp.mean(h, axis=1, keepdims=True)
    d_ = h - mu
    var = jnp.mean(d_ * d_, axis=1, keepdims=True)
    return d_ * lax.rsqrt(var + LN_EPS) * w + b


def _dense_body(x_ref, agg_ref, w1t_ref, ln1w_ref, ln1b_ref, w2t_ref,
                ln2w_ref, ln2b_ref, gnw_ref, gnb_ref, gnms_ref, out_ref):
    agg = jnp.concatenate([agg_ref[0, :N_NODES, :], agg_ref[1, :N_NODES, :]],
                          axis=1)
    h = x_ref[...] + agg
    h = jnp.dot(h, w1t_ref[...], preferred_element_type=jnp.float32)
    h = jnp.maximum(_ln(h, ln1w_ref[...], ln1b_ref[...]), 0.0)
    h = jnp.dot(h, w2t_ref[...], preferred_element_type=jnp.float32)
    h = jnp.maximum(_ln(h, ln2w_ref[...], ln2b_ref[...]), 0.0)
    mean = jnp.mean(h, axis=0, keepdims=True)
    o = h - gnms_ref[...] * mean
    var = jnp.mean(o * o, axis=0, keepdims=True)
    out_ref[...] = o * lax.rsqrt(var + GN_EPS) * gnw_ref[...] + gnb_ref[...]


_dense = pl.pallas_call(
    _dense_body,
    out_shape=jax.ShapeDtypeStruct((N_NODES, D), jnp.float32),
)


def kernel(x, edge_index, W1, ln1_w, ln1_b, W2, ln2_w, ln2_b,
           gn_weight, gn_bias, gn_mean_scale):
    xp = jnp.zeros((N_PAD, D), jnp.float32).at[:N_NODES].set(x)
    x2 = jnp.concatenate([xp[:, :DH], xp[:, DH:]], axis=0)
    pad = jnp.full((E_PAD - N_EDGES,), N_NODES, jnp.int32)
    src = jnp.concatenate([edge_index[0], pad]).reshape(NS * K, CHUNK)
    srcb = jnp.stack([src, src + N_PAD])
    dstp = jnp.concatenate([edge_index[1], pad]).reshape(NS * K, CHUNK)
    zeros = jnp.zeros((ROWS_PER_TILE, DH), jnp.float32)
    agg2 = _segment_sum_sc(x2, srcb, dstp, zeros)
    r = lambda v: v.reshape(1, D)
    return _dense(x, agg2, W1.T, r(ln1_w), r(ln1_b), W2.T, r(ln2_w),
                  r(ln2_b), r(gn_weight), r(gn_bias), r(gn_mean_scale))


# speedup vs baseline: 5.5286x; 1.0026x over previous
import functools

import jax
import jax.numpy as jnp
from jax import lax
from jax.experimental import pallas as pl
from jax.experimental.pallas import tpu as pltpu
from jax.experimental.pallas import tpu_sc as plsc

N_NODES = 10000
D = 128
DH = D // 2
N_EDGES = 320000
LN_EPS = 1e-5
GN_EPS = 1e-5

NC = 2
NS = 16
CHUNK = 128
K = 160
E_PAD = NS * K * CHUNK
N_PAD = 10112
ROWS_PER_TILE = N_PAD // NS
NBUF = 5
GAP = 3
OUT_S = 2

_mesh = plsc.VectorSubcoreMesh(core_axis_name="c", subcore_axis_name="s")


@functools.partial(
    pl.kernel,
    out_type=jax.ShapeDtypeStruct((NC, N_PAD, DH), jnp.float32),
    mesh=_mesh,
    compiler_params=pltpu.CompilerParams(use_tc_tiling_on_sc=False),
    scratch_types=[
        pltpu.VMEM((K, CHUNK), jnp.int32),
        pltpu.VMEM((K, CHUNK), jnp.int32),
        [pltpu.VMEM((CHUNK, DH), jnp.float32)] * NBUF,
        pltpu.VMEM_SHARED((N_PAD, DH), jnp.float32),
        pltpu.SemaphoreType.DMA,
        pltpu.SemaphoreType.DMA,
    ],
)
def _segment_sum_sc(x2_hbm, src_hbm, dst_hbm, zeros_hbm, out_hbm,
                    src_v, dst_v, rows, agg_sh, sem_g, sem_s):
    c = lax.axis_index("c")
    s = lax.axis_index("s")
    pltpu.sync_copy(zeros_hbm, agg_sh.at[pl.ds(s * ROWS_PER_TILE, ROWS_PER_TILE)])
    pltpu.sync_copy(src_hbm.at[c, pl.ds(s * K, K)], src_v)
    pltpu.sync_copy(dst_hbm.at[pl.ds(s * K, K)], dst_v)
    plsc.subcore_barrier()

    for b in range(GAP):
        pltpu.async_copy(x2_hbm.at[src_v.at[b]], rows[b], sem_g)

    def body(g, carry):
        j0 = g * NBUF
        for b in range(NBUF):
            j = j0 + b
            pltpu.make_async_copy(x2_hbm.at[src_v.at[j]], rows[b], sem_g).wait()
            pltpu.async_copy(rows[b], agg_sh.at[dst_v.at[j]], sem_s, add=True)

            @pl.when(j >= OUT_S)
            def _():
                pltpu.make_async_copy(rows[(b - OUT_S) % NBUF],
                                      agg_sh.at[dst_v.at[j - OUT_S]],
                                      sem_s).wait()

            @pl.when(j + GAP < K)
            def _():
                jn = j + GAP
                pltpu.async_copy(x2_hbm.at[src_v.at[jn]],
                                 rows[(b + GAP) % NBUF], sem_g)
        return carry

    lax.fori_loop(0, K // NBUF, body, 0)
    for t in range(OUT_S):
        jd = K - OUT_S + t
        pltpu.make_async_copy(rows[jd % NBUF],
                              agg_sh.at[dst_v.at[jd]], sem_s).wait()
    plsc.subcore_barrier()
    pltpu.sync_copy(agg_sh.at[pl.ds(s * ROWS_PER_TILE, ROWS_PER_TILE)],
                    out_hbm.at[c, pl.ds(s * ROWS_PER_TILE, ROWS_PER_TILE)])


def _ln(h, w, b):
    mu = jnp.mean(h, axis=1, keepdims=True)
    d_ = h - mu
    var = jnp.mean(d_ * d_, axis=1, keepdims=True)
    return d_ * lax.rsqrt(var + LN_EPS) * w + b


def _dense_body(x_ref, agg_ref, w1t_ref, ln1w_ref, ln1b_ref, w2t_ref,
                ln2w_ref, ln2b_ref, gnw_ref, gnb_ref, gnms_ref, out_ref):
    agg = jnp.concatenate([agg_ref[0, :N_NODES, :], agg_ref[1, :N_NODES, :]],
                          axis=1)
    h = x_ref[...] + agg
    h = jnp.dot(h, w1t_ref[...], preferred_element_type=jnp.float32)
    h = jnp.maximum(_ln(h, ln1w_ref[...], ln1b_ref[...]), 0.0)
    h = jnp.dot(h, w2t_ref[...], preferred_element_type=jnp.float32)
    h = jnp.maximum(_ln(h, ln2w_ref[...], ln2b_ref[...]), 0.0)
    mean = jnp.mean(h, axis=0, keepdims=True)
    o = h - gnms_ref[...] * mean
    var = jnp.mean(o * o, axis=0, keepdims=True)
    out_ref[...] = o * lax.rsqrt(var + GN_EPS) * gnw_ref[...] + gnb_ref[...]


_dense = pl.pallas_call(
    _dense_body,
    out_shape=jax.ShapeDtypeStruct((N_NODES, D), jnp.float32),
)


def kernel(x, edge_index, W1, ln1_w, ln1_b, W2, ln2_w, ln2_b,
           gn_weight, gn_bias, gn_mean_scale):
    xp = jnp.zeros((N_PAD, D), jnp.float32).at[:N_NODES].set(x)
    x2 = jnp.concatenate([xp[:, :DH], xp[:, DH:]], axis=0)
    pad = jnp.full((E_PAD - N_EDGES,), N_NODES, jnp.int32)
    src = jnp.concatenate([edge_index[0], pad]).reshape(NS * K, CHUNK)
    srcb = jnp.stack([src, src + N_PAD])
    dstp = jnp.concatenate([edge_index[1], pad]).reshape(NS * K, CHUNK)
    zeros = jnp.zeros((ROWS_PER_TILE, DH), jnp.float32)
    agg2 = _segment_sum_sc(x2, srcb, dstp, zeros)
    r = lambda v: v.reshape(1, D)
    return _dense(x, agg2, W1.T, r(ln1_w), r(ln1_b), W2.T, r(ln2_w),
                  r(ln2_b), r(gn_weight), r(gn_bias), r(gn_mean_scale))
